# baseline (speedup 1.0000x reference)
from functools import partial

import jax
import jax.numpy as jnp
from jax.experimental import pallas as pl
from jax.experimental.pallas import tpu as pltpu

_B_BLK = 32
_DT = jnp.bfloat16


def _conv3x3(a, offs, wcat, b):
    R, cin = a.shape
    L = R - offs[-1]
    cols = jnp.concatenate([a[o:o + L] for o in offs], axis=1)
    cols = jnp.concatenate(
        [cols, jnp.zeros((offs[-1], len(offs) * cin), a.dtype)], axis=0)
    z = jnp.dot(cols, wcat.astype(_DT),
                preferred_element_type=jnp.float32) + b
    return jnp.maximum(z, 0.0).astype(_DT)


def _pool2x2(z, d, C):
    R = z.shape[0]
    m = jnp.maximum(z[:R - d], z[d:])
    m = jnp.maximum(m[:-32], m[32:])
    m = jnp.concatenate([m, jnp.zeros((32 + d, C), z.dtype)], axis=0)
    m = m.reshape(-1, 2, 32, C)[:, 0]
    return m.reshape(R // 2, C)


def _conv1_pool1(xs, w1g, b1g):
    t = jnp.transpose(xs.astype(_DT).reshape(24, 1024), (1, 0))
    offs = [kh * 32 + kw for kh in range(3) for kw in range(3)]
    L = 1024 - offs[-1]
    cols = jnp.concatenate([t[o:o + L] for o in offs], axis=1)
    cols = jnp.concatenate(
        [cols, jnp.zeros((offs[-1], 216), t.dtype)], axis=0)
    z1g = jnp.maximum(
        jnp.dot(cols, w1g, preferred_element_type=jnp.float32) + b1g,
        0.0).astype(_DT)
    m = jnp.maximum(z1g[:-1], z1g[1:])
    m = jnp.maximum(m[:-32], m[32:])
    m = jnp.concatenate([m, jnp.zeros((33, 256), m.dtype)], axis=0)
    return m.reshape(16, 2, 32, 256)[:, 0].reshape(512, 256)


def _fused_kernel(x_ref, w1g_ref, b1g_ref, w2g_ref, b2g_ref, w3g_ref, b3g_ref,
                  wfg_ref, bfg_ref, wog_ref, bog_ref, o_ref, *, Bb):
    w1g = w1g_ref[...].astype(_DT)
    b1g = b1g_ref[...]
    ng = Bb // 4
    p1gs = [_conv1_pool1(x_ref[8 * s:8 * (s + 1)], w1g, b1g)
            for s in range(Bb // 8)]
    a2 = jnp.concatenate(
        [p[:, 128 * j:128 * (j + 1)] for p in p1gs for j in range(2)], axis=0)

    z2 = _conv3x3(a2, [kh * 32 + 2 * kw for kh in range(3) for kw in range(3)],
                  w2g_ref[...], b2g_ref[...])
    p2 = _pool2x2(z2, 2, 256)
    z3 = _conv3x3(p2, [kh * 32 + 4 * kw for kh in range(3) for kw in range(3)],
                  w3g_ref[...], b3g_ref[...])

    wf = wfg_ref[...].astype(_DT)
    acc = bfg_ref[...].astype(jnp.float32)
    R = ng * 256
    Lf = R - (3 * 32 + 4 * 3)
    for p in range(16):
        o = (p // 4) * 32 + (p % 4) * 4
        sl = z3[o:o + Lf]
        acc = acc + jnp.dot(sl, wf[p * 256:(p + 1) * 256],
                            preferred_element_type=jnp.float32)
    h1 = jnp.maximum(acc, 0.0).astype(_DT)
    out = jnp.dot(h1, wog_ref[...].astype(_DT),
                  preferred_element_type=jnp.float32) + bog_ref[...]
    out = jnp.concatenate(
        [out, jnp.zeros((R - Lf, 40), jnp.float32)], axis=0)
    o_ref[...] = out.reshape(ng, 256, 40)[:, 0:1, :]


def _const_spec(shape):
    n = len(shape)
    return pl.BlockSpec(tuple(shape), lambda i, _n=n: (0,) * _n)


def _blockdiag(w, n):
    t, c, o = w.shape
    eye = jnp.eye(n, dtype=w.dtype)
    wg = w[:, None, :, None, :] * eye[None, :, None, :, None]
    return wg.reshape(t * n * c, n * o)


def kernel(x, w1, b1, w2, b2, w3, b3, s1, s2, w_fc1, b_fc1, w_fc2, b_fc2):
    del s1, s2
    B = x.shape[0]
    xr = x.reshape(B, 3, 1024)
    Bb = _B_BLK
    w1g = _blockdiag(w1[:, :3, :], 8)
    b1g = jnp.tile(b1, (1, 8))
    w2g = _blockdiag(w2, 4)
    b2g = jnp.tile(b2, (1, 4))
    w3g = _blockdiag(w3, 4)
    b3g = jnp.tile(b3, (1, 4))
    wfg = _blockdiag(w_fc1.reshape(16, 64, 64), 4)
    bfg = jnp.tile(b_fc1, (1, 4))
    wog = _blockdiag(w_fc2[None], 4)[:, :]
    bog = jnp.tile(b_fc2, (1, 4))
    out = pl.pallas_call(
        partial(_fused_kernel, Bb=Bb),
        out_shape=jax.ShapeDtypeStruct((B // 4, 1, 40), jnp.float32),
        grid=(B // Bb,),
        out_specs=pl.BlockSpec((Bb // 4, 1, 40), lambda i: (i, 0, 0)),
        in_specs=[
            pl.BlockSpec((Bb, 3, 1024), lambda i: (i, 0, 0)),
            _const_spec(w1g.shape), _const_spec(b1g.shape),
            _const_spec(w2g.shape), _const_spec(b2g.shape),
            _const_spec(w3g.shape), _const_spec(b3g.shape),
            _const_spec(wfg.shape), _const_spec(bfg.shape),
            _const_spec(wog.shape), _const_spec(bog.shape),
        ],
        compiler_params=pltpu.CompilerParams(
            dimension_semantics=("parallel",)),
    )(xr, w1g, b1g, w2g, b2g, w3g, b3g, wfg, bfg, wog, bog)
    return out.reshape(B, 10)


# speedup vs baseline: 1.0164x; 1.0164x over previous
from functools import partial

import jax
import jax.numpy as jnp
from jax.experimental import pallas as pl
from jax.experimental.pallas import tpu as pltpu

_B_BLK = 16
_DT = jnp.bfloat16


def _conv3x3(a, offs, wcat, b):
    R, cin = a.shape
    L = R - offs[-1]
    cols = jnp.concatenate([a[o:o + L] for o in offs], axis=1)
    cols = jnp.concatenate(
        [cols, jnp.zeros((offs[-1], len(offs) * cin), a.dtype)], axis=0)
    z = jnp.dot(cols, wcat.astype(_DT),
                preferred_element_type=jnp.float32) + b
    return jnp.maximum(z, 0.0).astype(_DT)


def _pool2x2(z, d, C):
    R = z.shape[0]
    m = jnp.maximum(z[:R - d], z[d:])
    m = jnp.maximum(m[:-32], m[32:])
    m = jnp.concatenate([m, jnp.zeros((32 + d, C), z.dtype)], axis=0)
    m = m.reshape(-1, 2, 32, C)[:, 0]
    return m.reshape(R // 2, C)


def _conv1_pool1(xs, w1g, b1g):
    t = jnp.transpose(xs.astype(_DT).reshape(24, 1024), (1, 0))
    offs = [kh * 32 + kw for kh in range(3) for kw in range(3)]
    L = 1024 - offs[-1]
    cols = jnp.concatenate([t[o:o + L] for o in offs], axis=1)
    cols = jnp.concatenate(
        [cols, jnp.zeros((offs[-1], 216), t.dtype)], axis=0)
    z1g = jnp.maximum(
        jnp.dot(cols, w1g, preferred_element_type=jnp.float32) + b1g,
        0.0).astype(_DT)
    m = jnp.maximum(z1g[:-1], z1g[1:])
    m = jnp.maximum(m[:-32], m[32:])
    m = jnp.concatenate([m, jnp.zeros((33, 256), m.dtype)], axis=0)
    return m.reshape(16, 2, 32, 256)[:, 0].reshape(512, 256)


def _fused_kernel(x_ref, w1g_ref, b1g_ref, w2g_ref, b2g_ref, w3g_ref, b3g_ref,
                  wfg_ref, bfg_ref, wog_ref, bog_ref, o_ref, *, Bb):
    w1g = w1g_ref[...].astype(_DT)
    b1g = b1g_ref[...]
    ng = Bb // 4
    p1gs = [_conv1_pool1(x_ref[8 * s:8 * (s + 1)], w1g, b1g)
            for s in range(Bb // 8)]
    a2 = jnp.concatenate(
        [p[:, 128 * j:128 * (j + 1)] for p in p1gs for j in range(2)], axis=0)

    z2 = _conv3x3(a2, [kh * 32 + 2 * kw for kh in range(3) for kw in range(3)],
                  w2g_ref[...], b2g_ref[...])
    p2 = _pool2x2(z2, 2, 256)
    z3 = _conv3x3(p2, [kh * 32 + 4 * kw for kh in range(3) for kw in range(3)],
                  w3g_ref[...], b3g_ref[...])

    wf = wfg_ref[...].astype(_DT)
    acc = bfg_ref[...].astype(jnp.float32)
    R = ng * 256
    Lf = R - (3 * 32 + 4 * 3)
    for p in range(16):
        o = (p // 4) * 32 + (p % 4) * 4
        sl = z3[o:o + Lf]
        acc = acc + jnp.dot(sl, wf[p * 256:(p + 1) * 256],
                            preferred_element_type=jnp.float32)
    h1 = jnp.maximum(acc, 0.0).astype(_DT)
    out = jnp.dot(h1, wog_ref[...].astype(_DT),
                  preferred_element_type=jnp.float32) + bog_ref[...]
    out = jnp.concatenate(
        [out, jnp.zeros((R - Lf, 40), jnp.float32)], axis=0)
    o_ref[...] = out.reshape(ng, 256, 40)[:, 0:1, :]


def _const_spec(shape):
    n = len(shape)
    return pl.BlockSpec(tuple(shape), lambda i, _n=n: (0,) * _n)


def _blockdiag(w, n):
    t, c, o = w.shape
    eye = jnp.eye(n, dtype=w.dtype)
    wg = w[:, None, :, None, :] * eye[None, :, None, :, None]
    return wg.reshape(t * n * c, n * o)


def kernel(x, w1, b1, w2, b2, w3, b3, s1, s2, w_fc1, b_fc1, w_fc2, b_fc2):
    del s1, s2
    B = x.shape[0]
    xr = x.reshape(B, 3, 1024)
    Bb = _B_BLK
    w1g = _blockdiag(w1[:, :3, :], 8)
    b1g = jnp.tile(b1, (1, 8))
    w2g = _blockdiag(w2, 4)
    b2g = jnp.tile(b2, (1, 4))
    w3g = _blockdiag(w3, 4)
    b3g = jnp.tile(b3, (1, 4))
    wfg = _blockdiag(w_fc1.reshape(16, 64, 64), 4)
    bfg = jnp.tile(b_fc1, (1, 4))
    wog = _blockdiag(w_fc2[None], 4)[:, :]
    bog = jnp.tile(b_fc2, (1, 4))
    out = pl.pallas_call(
        partial(_fused_kernel, Bb=Bb),
        out_shape=jax.ShapeDtypeStruct((B // 4, 1, 40), jnp.float32),
        grid=(B // Bb,),
        out_specs=pl.BlockSpec((Bb // 4, 1, 40), lambda i: (i, 0, 0)),
        in_specs=[
            pl.BlockSpec((Bb, 3, 1024), lambda i: (i, 0, 0)),
            _const_spec(w1g.shape), _const_spec(b1g.shape),
            _const_spec(w2g.shape), _const_spec(b2g.shape),
            _const_spec(w3g.shape), _const_spec(b3g.shape),
            _const_spec(wfg.shape), _const_spec(bfg.shape),
            _const_spec(wog.shape), _const_spec(bog.shape),
        ],
        compiler_params=pltpu.CompilerParams(
            dimension_semantics=("parallel",)),
    )(xr, w1g, b1g, w2g, b2g, w3g, b3g, wfg, bfg, wog, bog)
    return out.reshape(B, 10)
